# baseline (speedup 1.0000x reference)
import functools

import jax
import jax.numpy as jnp
from jax import lax
from jax.experimental import pallas as pl
from jax.experimental.pallas import tpu as pltpu
from jax.experimental.pallas import tpu_sc as plsc

NUM_FEATS = 5
EMB_DIM = 128
N_NODES = 100000
OUT_DIM = NUM_FEATS * EMB_DIM

NODE_CHUNK = 80
CHUNK = NODE_CHUNK * NUM_FEATS
N_CHUNKS = N_NODES // NODE_CHUNK
LANES = 16
VPR = EMB_DIM // LANES
NGRP = NODE_CHUNK // LANES


def _make_sc_expand():
    info = plsc.get_sparse_core_info()
    nw = info.num_cores * info.num_subcores
    k_max = -(-N_CHUNKS // nw)
    assert k_max % 2 == 0

    mesh = plsc.VectorSubcoreMesh(core_axis_name="c", subcore_axis_name="s")

    @functools.partial(
        pl.kernel,
        mesh=mesh,
        out_type=jax.ShapeDtypeStruct((N_NODES, OUT_DIM), jnp.float32),
        scratch_types=[
            pltpu.VMEM((CHUNK,), jnp.int32),
            pltpu.VMEM((CHUNK,), jnp.int32),
            pltpu.VMEM((2, NUM_FEATS, NODE_CHUNK, EMB_DIM), jnp.float32),
            pltpu.VMEM((NUM_FEATS, EMB_DIM), jnp.float32),
            pltpu.SemaphoreType.DMA,
            pltpu.SemaphoreType.DMA,
        ],
    )
    def sc_expand(table_hbm, idx_hbm, out_hbm, idx_v0, idx_v1, rows_v, w_t, sem_i, sem_o):
        wid = lax.axis_index("s") * info.num_cores + lax.axis_index("c")
        idx_bufs = (idx_v0, idx_v1)

        pltpu.sync_copy(table_hbm, w_t)

        def idx_copy(k, b):
            ci = wid + k * nw
            return pltpu.make_async_copy(idx_hbm.at[ci], idx_bufs[b], sem_i)

        def expand(b):
            for f in range(NUM_FEATS):

                def gbody(g, carry, f=f, b=b):
                    n0 = g * LANES
                    v = idx_bufs[b][pl.ds(f * NODE_CHUNK + n0, LANES)]
                    for m in range(LANES):
                        i = v[m]
                        for j in range(VPR):
                            rows_v[b, f, n0 + m, pl.ds(j * LANES, LANES)] = w_t[
                                i, pl.ds(j * LANES, LANES)
                            ]
                    return carry

                lax.fori_loop(0, NGRP, gbody, None)

        def scatter_subs(k, b):
            ci = wid + k * nw
            return [
                pltpu.make_async_copy(
                    rows_v.at[b, f],
                    out_hbm.at[
                        pl.ds(ci * NODE_CHUNK, NODE_CHUNK), pl.ds(f * EMB_DIM, EMB_DIM)
                    ],
                    sem_o,
                )
                for f in range(NUM_FEATS)
            ]

        def live(k):
            return wid + k * nw < N_CHUNKS

        @pl.when(live(0))
        def _():
            idx_copy(0, 0).start()

        def pair_body(k2, carry):
            for b in (0, 1):
                k = 2 * k2 + b

                @pl.when(jnp.logical_and(live(k), k >= 2))
                def _(k=k, b=b):
                    for d in scatter_subs(k - 2, b):
                        d.wait()

                @pl.when(live(k))
                def _(k=k, b=b):
                    idx_copy(k, b).wait()

                @pl.when(jnp.logical_and(live(k + 1), k + 1 < k_max))
                def _(k=k, b=b):
                    idx_copy(k + 1, 1 - b).start()

                @pl.when(live(k))
                def _(k=k, b=b):
                    expand(b)
                    for d in scatter_subs(k, b):
                        d.start()

            return carry

        lax.fori_loop(0, k_max // 2, pair_body, None)

        for k in (k_max - 2, k_max - 1):

            @pl.when(live(k))
            def _(k=k):
                for d in scatter_subs(k, k % 2):
                    d.wait()

    return sc_expand


_sc_expand = _make_sc_expand()


def kernel(x, W):
    idx = (
        x.astype(jnp.int32)
        .T.reshape(NUM_FEATS, N_CHUNKS, NODE_CHUNK)
        .transpose(1, 0, 2)
        .reshape(N_CHUNKS, CHUNK)
    )
    return _sc_expand(W, idx)


# speedup vs baseline: 3.8425x; 3.8425x over previous
import functools

import jax
import jax.numpy as jnp
from jax import lax
from jax.experimental import pallas as pl
from jax.experimental.pallas import tpu as pltpu
from jax.experimental.pallas import tpu_sc as plsc

NUM_FEATS = 5
EMB_DIM = 128
N_NODES = 100000
OUT_DIM = NUM_FEATS * EMB_DIM

NODE_CHUNK = 80
CHUNK = NODE_CHUNK * NUM_FEATS
N_CHUNKS = N_NODES // NODE_CHUNK
NSUB = 5
SUB = CHUNK // NSUB


def _make_sc_gather():
    info = plsc.get_sparse_core_info()
    nw = info.num_cores * info.num_subcores
    k_max = -(-N_CHUNKS // nw)

    mesh = plsc.VectorSubcoreMesh(core_axis_name="c", subcore_axis_name="s")

    @functools.partial(
        pl.kernel,
        mesh=mesh,
        out_type=jax.ShapeDtypeStruct((N_NODES, OUT_DIM), jnp.float32),
        scratch_types=[
            pltpu.VMEM((CHUNK,), jnp.int32),
            pltpu.VMEM((CHUNK,), jnp.int32),
            pltpu.VMEM((2, NUM_FEATS, NODE_CHUNK, EMB_DIM), jnp.float32),
            pltpu.VMEM_SHARED((16 * NUM_FEATS, EMB_DIM), jnp.float32),
            pltpu.SemaphoreType.DMA,
            pltpu.SemaphoreType.DMA,
            pltpu.SemaphoreType.DMA,
        ],
    )
    def sc_gather(
        table_hbm, idx_hbm, out_hbm, idx_v0, idx_v1, rows_v, table_sh, sem_i, sem_g, sem_o
    ):
        wid = lax.axis_index("s") * info.num_cores + lax.axis_index("c")
        idx_bufs = (idx_v0, idx_v1)

        sid = lax.axis_index("s")
        pltpu.sync_copy(table_hbm, table_sh.at[pl.ds(sid * NUM_FEATS, NUM_FEATS)])

        plsc.subcore_barrier()

        def idx_copy(k, b):
            ci = wid + k * nw
            return pltpu.make_async_copy(idx_hbm.at[ci], idx_bufs[b], sem_i)

        def gather_subs(k, b):
            return [
                pltpu.make_async_copy(
                    table_sh.at[idx_bufs[b].at[pl.ds(f * NODE_CHUNK, NODE_CHUNK)]],
                    rows_v.at[b, f],
                    sem_g,
                )
                for f in range(NUM_FEATS)
            ]

        def scatter_subs(k, b):
            ci = wid + k * nw
            return [
                pltpu.make_async_copy(
                    rows_v.at[b, f],
                    out_hbm.at[
                        pl.ds(ci * NODE_CHUNK, NODE_CHUNK), pl.ds(f * EMB_DIM, EMB_DIM)
                    ],
                    sem_o,
                )
                for f in range(NUM_FEATS)
            ]

        def live(k):
            return wid + k * nw < N_CHUNKS

        @pl.when(live(0))
        def _():
            idx_copy(0, 0).start()

        for k in range(k_max):
            b = k % 2

            if k >= 2:

                @pl.when(live(k))
                def _(k=k, b=b):
                    for d in scatter_subs(k - 2, b):
                        d.wait()

            @pl.when(live(k))
            def _(k=k, b=b):
                idx_copy(k, b).wait()
                for d in gather_subs(k, b):
                    d.start()

            if k + 1 < k_max:

                @pl.when(live(k + 1))
                def _(k=k, b=b):
                    idx_copy(k + 1, 1 - b).start()

            @pl.when(live(k))
            def _(k=k, b=b):
                for d in gather_subs(k, b):
                    d.wait()
                for d in scatter_subs(k, b):
                    d.start()

        for k in range(max(0, k_max - 2), k_max):

            @pl.when(live(k))
            def _(k=k):
                for d in scatter_subs(k, k % 2):
                    d.wait()

    return sc_gather


_sc_gather = _make_sc_gather()


def kernel(x, W):
    idx = (
        x.astype(jnp.int32)
        .T.reshape(NUM_FEATS, N_CHUNKS, NODE_CHUNK)
        .transpose(1, 0, 2)
        .reshape(N_CHUNKS, CHUNK)
    )
    sub = ((jnp.arange(N_CHUNKS, dtype=jnp.int32) % 32) // 2) * NUM_FEATS
    idx = idx + sub[:, None]
    return _sc_gather(W, idx)
